# baseline (speedup 1.0000x reference)
import functools

import jax
import jax.numpy as jnp
from jax import lax
from jax.experimental import pallas as pl
from jax.experimental.pallas import tpu as pltpu
from jax.experimental.pallas import tpu_sc as plsc

N_NODES = 10000
N_PAD = 10240
E_EDGES = 320000
CHUNK = 128
NW = 32
NCH_W = 80
NUM_CHUNKS = NW * NCH_W
NBUF = 4
HID = 64
ROWS_PER_TILE = N_PAD // 16


def _sc_segment_sum(with_cnt):
  mesh = plsc.VectorSubcoreMesh(core_axis_name="c", subcore_axis_name="s")
  out_type = [jax.ShapeDtypeStruct((2, N_PAD, HID), jnp.float32)]
  scratch = [
      pltpu.VMEM((NCH_W, CHUNK), jnp.int32),
      pltpu.VMEM((NCH_W, CHUNK), jnp.int32),
      pltpu.VMEM((NBUF, CHUNK, HID), jnp.float32),
      pltpu.VMEM_SHARED((N_PAD, HID), jnp.float32),
  ] + [pltpu.SemaphoreType.DMA] * NBUF
  if with_cnt:
    out_type.append(jax.ShapeDtypeStruct((2, N_PAD, 16), jnp.float32))
    scratch += [
        pltpu.VMEM((CHUNK, 16), jnp.float32),
        pltpu.VMEM_SHARED((N_PAD, 16), jnp.float32),
    ]

  def body(*refs):
    if with_cnt:
      (y_hbm, src_hbm, dst_hbm, zrow_hbm, z16_hbm, ones_hbm,
       p_hbm, c_hbm,
       sidx, didx, rows, acc, s0, s1, s2, s3, ones_v, cacc) = refs
    else:
      (y_hbm, src_hbm, dst_hbm, zrow_hbm,
       p_hbm,
       sidx, didx, rows, acc, s0, s1, s2, s3) = refs
    sems = (s0, s1, s2, s3)

    cid = lax.axis_index("c")
    sid = lax.axis_index("s")
    wid = sid * 2 + cid

    for j in range(ROWS_PER_TILE // CHUNK):
      base = (sid * (ROWS_PER_TILE // CHUNK) + j) * CHUNK
      pltpu.sync_copy(zrow_hbm, acc.at[pl.ds(base, CHUNK)])
      if with_cnt:
        pltpu.sync_copy(z16_hbm, cacc.at[pl.ds(base, CHUNK)])
    pltpu.sync_copy(src_hbm.at[pl.ds(wid * NCH_W, NCH_W)], sidx)
    pltpu.sync_copy(dst_hbm.at[pl.ds(wid * NCH_W, NCH_W)], didx)
    if with_cnt:
      pltpu.sync_copy(ones_hbm, ones_v)
    plsc.subcore_barrier()

    def step(j, carry):
      pltpu.async_copy(y_hbm.at[sidx.at[j]], rows.at[0], sems[0]).wait()
      pltpu.sync_copy(rows.at[0], acc.at[didx.at[j]], add=True)
      if with_cnt:
        pltpu.sync_copy(ones_v, cacc.at[didx.at[j]], add=True)
      return carry

    lax.fori_loop(0, NCH_W, step, 0)
    plsc.subcore_barrier()

    rbase = sid * ROWS_PER_TILE
    pltpu.sync_copy(acc.at[pl.ds(rbase, ROWS_PER_TILE)],
                    p_hbm.at[cid, pl.ds(rbase, ROWS_PER_TILE)])
    if with_cnt:
      pltpu.sync_copy(cacc.at[pl.ds(rbase, ROWS_PER_TILE)],
                      c_hbm.at[cid, pl.ds(rbase, ROWS_PER_TILE)])

  return pl.kernel(body, out_type=tuple(out_type), mesh=mesh,
                   scratch_types=tuple(scratch),
                   compiler_params=pltpu.CompilerParams(
                       use_tc_tiling_on_sc=False))


_seg_sum_cnt = _sc_segment_sum(with_cnt=True)
_seg_sum = _sc_segment_sum(with_cnt=False)


def _dot(a, b):
  return jnp.dot(a, b, preferred_element_type=jnp.float32)


def _tc0_body(x_ref, wl_ref, wr_ref, bl_ref, y_ref, r_ref):
  x = x_ref[...]
  y_ref[...] = _dot(x, wl_ref[...])
  r_ref[...] = _dot(x, wr_ref[...]) + bl_ref[...]


def _combine(p_ref, c_ref, r_ref, g_ref, b_ref):
  agg = p_ref[0, :N_NODES, :] + p_ref[1, :N_NODES, :]
  cnt = c_ref[0, :N_NODES, 0:1] + c_ref[1, :N_NODES, 0:1]
  pre = agg / jnp.maximum(cnt, 1.0) + r_ref[...]
  mu = jnp.mean(pre, axis=0, keepdims=True)
  var = jnp.mean((pre - mu) * (pre - mu), axis=0, keepdims=True)
  h = g_ref[...] * (pre - mu) / jnp.sqrt(var + 1e-5) + b_ref[...]
  return jnp.maximum(h, 0.0)


def _tc1_body(p_ref, c_ref, r_ref, g_ref, b_ref, wl_ref, wr_ref, bl_ref,
              x1_ref, y_ref, r2_ref):
  h = _combine(p_ref, c_ref, r_ref, g_ref, b_ref)
  x1_ref[...] = h
  y_ref[...] = _dot(h, wl_ref[...])
  r2_ref[...] = _dot(h, wr_ref[...]) + bl_ref[...]


def _tc2_body(p_ref, c_ref, r_ref, x1_ref, g_ref, b_ref, wl_ref, wr_ref,
              bl_ref, y_ref, r2_ref):
  h = _combine(p_ref, c_ref, r_ref, g_ref, b_ref)
  x2 = x1_ref[...] + 0.3 * h
  y_ref[...] = _dot(x2, wl_ref[...])
  r2_ref[...] = _dot(x2, wr_ref[...]) + bl_ref[...]


def _tc3_body(p_ref, c_ref, r_ref, g_ref, b_ref, out_ref):
  out_ref[...] = _combine(p_ref, c_ref, r_ref, g_ref, b_ref)


_f32 = jnp.float32


def _tc_call(body, out_shapes, *args):
  return pl.pallas_call(
      body,
      out_shape=[jax.ShapeDtypeStruct(s, _f32) for s in out_shapes],
  )(*args)


@jax.jit
def kernel(x, edge_index, W_l0, b_l0, W_r0, bn_g0, bn_b0, W_l1, b_l1, W_r1,
           bn_g1, bn_b1, W_l2, b_l2, W_r2, bn_g2, bn_b2):
  n_extra = NUM_CHUNKS * CHUNK - E_EDGES
  pad_dst = N_NODES + jnp.arange(n_extra, dtype=jnp.int32) % (N_PAD - N_NODES)
  src = jnp.concatenate(
      [edge_index[0], jnp.zeros((n_extra,), jnp.int32)]).reshape(
          NUM_CHUNKS, CHUNK)
  dst = jnp.concatenate([edge_index[1], pad_dst]).reshape(NUM_CHUNKS, CHUNK)
  zrow = jnp.zeros((CHUNK, HID), _f32)
  z16 = jnp.zeros((CHUNK, 16), _f32)
  ones16 = jnp.zeros((CHUNK, 16), _f32).at[:, 0].set(1.0)

  y0, r0 = _tc_call(_tc0_body, [(N_NODES, HID), (N_NODES, HID)],
                    x, W_l0, W_r0, b_l0.reshape(1, HID))
  p0, c0 = _seg_sum_cnt(y0, src, dst, zrow, z16, ones16)
  x1, y1, r1 = _tc_call(
      _tc1_body, [(N_NODES, HID), (N_NODES, HID), (N_NODES, HID)],
      p0, c0, r0, bn_g0.reshape(1, HID), bn_b0.reshape(1, HID),
      W_l1, W_r1, b_l1.reshape(1, HID))
  (p1,) = _seg_sum(y1, src, dst, zrow)
  y2, r2 = _tc_call(
      _tc2_body, [(N_NODES, HID), (N_NODES, HID)],
      p1, c0, r1, x1, bn_g1.reshape(1, HID), bn_b1.reshape(1, HID),
      W_l2, W_r2, b_l2.reshape(1, HID))
  (p2,) = _seg_sum(y2, src, dst, zrow)
  (out,) = _tc_call(
      _tc3_body, [(N_NODES, HID)],
      p2, c0, r2, bn_g2.reshape(1, HID), bn_b2.reshape(1, HID))
  return out


# speedup vs baseline: 1.2172x; 1.2172x over previous
import functools

import jax
import jax.numpy as jnp
from jax import lax
from jax.experimental import pallas as pl
from jax.experimental.pallas import tpu as pltpu
from jax.experimental.pallas import tpu_sc as plsc

N_NODES = 10000
N_PAD = 10240
E_EDGES = 320000
CHUNK = 128
NW = 32
NCH_W = 80
NUM_CHUNKS = NW * NCH_W
NBUF = 4
HID = 64
ROWS_PER_TILE = N_PAD // 16


def _sc_segment_sum(with_cnt):
  mesh = plsc.VectorSubcoreMesh(core_axis_name="c", subcore_axis_name="s")
  out_type = [jax.ShapeDtypeStruct((2, N_PAD, HID), jnp.float32)]
  scratch = (
      [pltpu.VMEM((CHUNK,), jnp.int32) for _ in range(NBUF)]
      + [pltpu.VMEM((CHUNK,), jnp.int32) for _ in range(NBUF)]
      + [pltpu.VMEM((NBUF, CHUNK, HID), jnp.float32),
         pltpu.VMEM_SHARED((N_PAD, HID), jnp.float32)]
      + [pltpu.SemaphoreType.DMA] * NBUF)
  if with_cnt:
    out_type.append(jax.ShapeDtypeStruct((2, N_PAD, 16), jnp.float32))
    scratch += [
        pltpu.VMEM((CHUNK, 16), jnp.float32),
        pltpu.VMEM_SHARED((N_PAD, 16), jnp.float32),
    ]

  def body(*refs):
    if with_cnt:
      (y_hbm, src_hbm, dst_hbm, zrow_hbm, z16_hbm, ones_hbm,
       p_hbm, c_hbm,
       si0, si1, si2, si3, di0, di1, di2, di3,
       rows, acc, s0, s1, s2, s3, ones_v, cacc) = refs
    else:
      (y_hbm, src_hbm, dst_hbm, zrow_hbm,
       p_hbm,
       si0, si1, si2, si3, di0, di1, di2, di3,
       rows, acc, s0, s1, s2, s3) = refs
    sidx = (si0, si1, si2, si3)
    didx = (di0, di1, di2, di3)
    sems = (s0, s1, s2, s3)

    cid = lax.axis_index("c")
    sid = lax.axis_index("s")
    wid = sid * 2 + cid

    for j in range(ROWS_PER_TILE // CHUNK):
      base = (sid * (ROWS_PER_TILE // CHUNK) + j) * CHUNK
      pltpu.sync_copy(zrow_hbm, acc.at[pl.ds(base, CHUNK)])
      if with_cnt:
        pltpu.sync_copy(z16_hbm, cacc.at[pl.ds(base, CHUNK)])
    if with_cnt:
      pltpu.sync_copy(ones_hbm, ones_v)
    plsc.subcore_barrier()

    def load_and_start(b, j):
      ebase = (wid * NCH_W + j) * CHUNK
      pltpu.sync_copy(src_hbm.at[pl.ds(ebase, CHUNK)], sidx[b])
      pltpu.sync_copy(dst_hbm.at[pl.ds(ebase, CHUNK)], didx[b])
      pltpu.async_copy(y_hbm.at[sidx[b]], rows.at[b], sems[b])

    def consume(b):
      pltpu.make_async_copy(y_hbm.at[sidx[b]], rows.at[b], sems[b]).wait()
      pltpu.sync_copy(rows.at[b], acc.at[didx[b]], add=True)
      if with_cnt:
        pltpu.sync_copy(ones_v, cacc.at[didx[b]], add=True)

    for b in range(NBUF):
      load_and_start(b, b)

    def outer(o, carry):
      for b in range(NBUF):
        j = o * NBUF + b
        consume(b)
        load_and_start(b, j + NBUF)
      return carry

    lax.fori_loop(0, (NCH_W - NBUF) // NBUF, outer, 0)
    for b in range(NBUF):
      consume(b)
    plsc.subcore_barrier()

    rbase = sid * ROWS_PER_TILE
    pltpu.sync_copy(acc.at[pl.ds(rbase, ROWS_PER_TILE)],
                    p_hbm.at[cid, pl.ds(rbase, ROWS_PER_TILE)])
    if with_cnt:
      pltpu.sync_copy(cacc.at[pl.ds(rbase, ROWS_PER_TILE)],
                      c_hbm.at[cid, pl.ds(rbase, ROWS_PER_TILE)])

  return pl.kernel(body, out_type=tuple(out_type), mesh=mesh,
                   scratch_types=tuple(scratch),
                   compiler_params=pltpu.CompilerParams(
                       use_tc_tiling_on_sc=False))


_seg_sum_cnt = _sc_segment_sum(with_cnt=True)
_seg_sum = _sc_segment_sum(with_cnt=False)


def _dot(a, b):
  return jnp.dot(a, b, preferred_element_type=jnp.float32)


def _tc0_body(x_ref, wl_ref, wr_ref, bl_ref, y_ref, r_ref):
  x = x_ref[...]
  y_ref[...] = _dot(x, wl_ref[...])
  r_ref[...] = _dot(x, wr_ref[...]) + bl_ref[...]


def _combine(p_ref, c_ref, r_ref, g_ref, b_ref):
  agg = p_ref[0, :N_NODES, :] + p_ref[1, :N_NODES, :]
  cnt = c_ref[0, :N_NODES, 0:1] + c_ref[1, :N_NODES, 0:1]
  pre = agg / jnp.maximum(cnt, 1.0) + r_ref[...]
  mu = jnp.mean(pre, axis=0, keepdims=True)
  var = jnp.mean((pre - mu) * (pre - mu), axis=0, keepdims=True)
  h = g_ref[...] * (pre - mu) / jnp.sqrt(var + 1e-5) + b_ref[...]
  return jnp.maximum(h, 0.0)


def _tc1_body(p_ref, c_ref, r_ref, g_ref, b_ref, wl_ref, wr_ref, bl_ref,
              x1_ref, y_ref, r2_ref):
  h = _combine(p_ref, c_ref, r_ref, g_ref, b_ref)
  x1_ref[...] = h
  y_ref[...] = _dot(h, wl_ref[...])
  r2_ref[...] = _dot(h, wr_ref[...]) + bl_ref[...]


def _tc2_body(p_ref, c_ref, r_ref, x1_ref, g_ref, b_ref, wl_ref, wr_ref,
              bl_ref, y_ref, r2_ref):
  h = _combine(p_ref, c_ref, r_ref, g_ref, b_ref)
  x2 = x1_ref[...] + 0.3 * h
  y_ref[...] = _dot(x2, wl_ref[...])
  r2_ref[...] = _dot(x2, wr_ref[...]) + bl_ref[...]


def _tc3_body(p_ref, c_ref, r_ref, g_ref, b_ref, out_ref):
  out_ref[...] = _combine(p_ref, c_ref, r_ref, g_ref, b_ref)


_f32 = jnp.float32


def _tc_call(body, out_shapes, *args):
  return pl.pallas_call(
      body,
      out_shape=[jax.ShapeDtypeStruct(s, _f32) for s in out_shapes],
  )(*args)


@jax.jit
def kernel(x, edge_index, W_l0, b_l0, W_r0, bn_g0, bn_b0, W_l1, b_l1, W_r1,
           bn_g1, bn_b1, W_l2, b_l2, W_r2, bn_g2, bn_b2):
  n_extra = NUM_CHUNKS * CHUNK - E_EDGES
  pad_dst = N_NODES + jnp.arange(n_extra, dtype=jnp.int32) % (N_PAD - N_NODES)
  src = jnp.concatenate([edge_index[0], jnp.zeros((n_extra,), jnp.int32)])
  dst = jnp.concatenate([edge_index[1], pad_dst])
  zrow = jnp.zeros((CHUNK, HID), _f32)
  z16 = jnp.zeros((CHUNK, 16), _f32)
  ones16 = jnp.zeros((CHUNK, 16), _f32).at[:, 0].set(1.0)

  y0, r0 = _tc_call(_tc0_body, [(N_NODES, HID), (N_NODES, HID)],
                    x, W_l0, W_r0, b_l0.reshape(1, HID))
  p0, c0 = _seg_sum_cnt(y0, src, dst, zrow, z16, ones16)
  x1, y1, r1 = _tc_call(
      _tc1_body, [(N_NODES, HID), (N_NODES, HID), (N_NODES, HID)],
      p0, c0, r0, bn_g0.reshape(1, HID), bn_b0.reshape(1, HID),
      W_l1, W_r1, b_l1.reshape(1, HID))
  (p1,) = _seg_sum(y1, src, dst, zrow)
  y2, r2 = _tc_call(
      _tc2_body, [(N_NODES, HID), (N_NODES, HID)],
      p1, c0, r1, x1, bn_g1.reshape(1, HID), bn_b1.reshape(1, HID),
      W_l2, W_r2, b_l2.reshape(1, HID))
  (p2,) = _seg_sum(y2, src, dst, zrow)
  (out,) = _tc_call(
      _tc3_body, [(N_NODES, HID)],
      p2, c0, r2, bn_g2.reshape(1, HID), bn_b2.reshape(1, HID))
  return out


# speedup vs baseline: 1.4153x; 1.1627x over previous
import functools

import jax
import jax.numpy as jnp
from jax import lax
from jax.experimental import pallas as pl
from jax.experimental.pallas import tpu as pltpu
from jax.experimental.pallas import tpu_sc as plsc

N_NODES = 10000
N_PAD = 10240
E_EDGES = 320000
CHUNK = 128
NW = 32
NUM_CHUNKS = E_EDGES // CHUNK
HID = 64
ROWS_PER_TILE = N_PAD // 16


def _sc_segment_sum(with_cnt):
  mesh = plsc.VectorSubcoreMesh(core_axis_name="c", subcore_axis_name="s")
  out_type = [jax.ShapeDtypeStruct((2, N_PAD, HID), jnp.float32)]
  scratch = [
      pltpu.VMEM((CHUNK,), jnp.int32),
      pltpu.VMEM((CHUNK,), jnp.int32),
      pltpu.VMEM((CHUNK, HID), jnp.float32),
      pltpu.VMEM_SHARED((N_PAD, HID), jnp.float32),
      pltpu.SemaphoreType.DMA,
  ]
  if with_cnt:
    out_type.append(jax.ShapeDtypeStruct((2, N_PAD, 16), jnp.float32))
    scratch += [
        pltpu.VMEM((CHUNK, 16), jnp.float32),
        pltpu.VMEM_SHARED((N_PAD, 16), jnp.float32),
    ]

  def body(*refs):
    if with_cnt:
      (y_hbm, src_hbm, dst_hbm, zrow_hbm, z16_hbm, ones_hbm,
       p_hbm, c_hbm,
       sidx, didx, rows, acc, sem, ones_v, cacc) = refs
    else:
      (y_hbm, src_hbm, dst_hbm, zrow_hbm,
       p_hbm,
       sidx, didx, rows, acc, sem) = refs

    cid = lax.axis_index("c")
    sid = lax.axis_index("s")
    wid = sid * 2 + cid

    for j in range(ROWS_PER_TILE // CHUNK):
      base = (sid * (ROWS_PER_TILE // CHUNK) + j) * CHUNK
      pltpu.sync_copy(zrow_hbm, acc.at[pl.ds(base, CHUNK)])
      if with_cnt:
        pltpu.sync_copy(z16_hbm, cacc.at[pl.ds(base, CHUNK)])
    if with_cnt:
      pltpu.sync_copy(ones_hbm, ones_v)
    plsc.subcore_barrier()

    nfull = NUM_CHUNKS // NW
    nrem = NUM_CHUNKS - nfull * NW
    trips = jnp.where(wid < nrem, nfull + 1, nfull)

    def step(k, carry):
      ebase = (wid + k * NW) * CHUNK
      pltpu.sync_copy(src_hbm.at[pl.ds(ebase, CHUNK)], sidx)
      pltpu.sync_copy(dst_hbm.at[pl.ds(ebase, CHUNK)], didx)
      pltpu.async_copy(y_hbm.at[sidx], rows, sem).wait()
      pltpu.sync_copy(rows, acc.at[didx], add=True)
      if with_cnt:
        pltpu.sync_copy(ones_v, cacc.at[didx], add=True)
      return carry

    lax.fori_loop(0, trips, step, 0)
    plsc.subcore_barrier()

    rbase = sid * ROWS_PER_TILE
    pltpu.sync_copy(acc.at[pl.ds(rbase, ROWS_PER_TILE)],
                    p_hbm.at[cid, pl.ds(rbase, ROWS_PER_TILE)])
    if with_cnt:
      pltpu.sync_copy(cacc.at[pl.ds(rbase, ROWS_PER_TILE)],
                      c_hbm.at[cid, pl.ds(rbase, ROWS_PER_TILE)])

  return pl.kernel(body, out_type=tuple(out_type), mesh=mesh,
                   scratch_types=tuple(scratch),
                   compiler_params=pltpu.CompilerParams(
                       use_tc_tiling_on_sc=False))


_seg_sum_cnt = _sc_segment_sum(with_cnt=True)
_seg_sum = _sc_segment_sum(with_cnt=False)


def _dot(a, b):
  return jnp.dot(a, b, preferred_element_type=jnp.float32)


def _tc0_body(x_ref, wl_ref, wr_ref, bl_ref, y_ref, r_ref):
  x = x_ref[...]
  y_ref[...] = _dot(x, wl_ref[...])
  r_ref[...] = _dot(x, wr_ref[...]) + bl_ref[...]


def _combine(p_ref, c_ref, r_ref, g_ref, b_ref):
  agg = p_ref[0, :N_NODES, :] + p_ref[1, :N_NODES, :]
  cnt = c_ref[0, :N_NODES, 0:1] + c_ref[1, :N_NODES, 0:1]
  pre = agg / jnp.maximum(cnt, 1.0) + r_ref[...]
  mu = jnp.mean(pre, axis=0, keepdims=True)
  var = jnp.mean((pre - mu) * (pre - mu), axis=0, keepdims=True)
  h = g_ref[...] * (pre - mu) / jnp.sqrt(var + 1e-5) + b_ref[...]
  return jnp.maximum(h, 0.0)


def _tc1_body(p_ref, c_ref, r_ref, g_ref, b_ref, wl_ref, wr_ref, bl_ref,
              x1_ref, y_ref, r2_ref):
  h = _combine(p_ref, c_ref, r_ref, g_ref, b_ref)
  x1_ref[...] = h
  y_ref[...] = _dot(h, wl_ref[...])
  r2_ref[...] = _dot(h, wr_ref[...]) + bl_ref[...]


def _tc2_body(p_ref, c_ref, r_ref, x1_ref, g_ref, b_ref, wl_ref, wr_ref,
              bl_ref, y_ref, r2_ref):
  h = _combine(p_ref, c_ref, r_ref, g_ref, b_ref)
  x2 = x1_ref[...] + 0.3 * h
  y_ref[...] = _dot(x2, wl_ref[...])
  r2_ref[...] = _dot(x2, wr_ref[...]) + bl_ref[...]


def _tc3_body(p_ref, c_ref, r_ref, g_ref, b_ref, out_ref):
  out_ref[...] = _combine(p_ref, c_ref, r_ref, g_ref, b_ref)


_f32 = jnp.float32


def _tc_call(body, out_shapes, *args):
  return pl.pallas_call(
      body,
      out_shape=[jax.ShapeDtypeStruct(s, _f32) for s in out_shapes],
  )(*args)


@jax.jit
def kernel(x, edge_index, W_l0, b_l0, W_r0, bn_g0, bn_b0, W_l1, b_l1, W_r1,
           bn_g1, bn_b1, W_l2, b_l2, W_r2, bn_g2, bn_b2):
  src = edge_index[0]
  dst = edge_index[1]
  zrow = jnp.zeros((CHUNK, HID), _f32)
  z16 = jnp.zeros((CHUNK, 16), _f32)
  ones16 = jnp.zeros((CHUNK, 16), _f32).at[:, 0].set(1.0)

  y0, r0 = _tc_call(_tc0_body, [(N_NODES, HID), (N_NODES, HID)],
                    x, W_l0, W_r0, b_l0.reshape(1, HID))
  p0, c0 = _seg_sum_cnt(y0, src, dst, zrow, z16, ones16)
  x1, y1, r1 = _tc_call(
      _tc1_body, [(N_NODES, HID), (N_NODES, HID), (N_NODES, HID)],
      p0, c0, r0, bn_g0.reshape(1, HID), bn_b0.reshape(1, HID),
      W_l1, W_r1, b_l1.reshape(1, HID))
  (p1,) = _seg_sum(y1, src, dst, zrow)
  y2, r2 = _tc_call(
      _tc2_body, [(N_NODES, HID), (N_NODES, HID)],
      p1, c0, r1, x1, bn_g1.reshape(1, HID), bn_b1.reshape(1, HID),
      W_l2, W_r2, b_l2.reshape(1, HID))
  (p2,) = _seg_sum(y2, src, dst, zrow)
  (out,) = _tc_call(
      _tc3_body, [(N_NODES, HID)],
      p2, c0, r2, bn_g2.reshape(1, HID), bn_b2.reshape(1, HID))
  return out
